# baseline (speedup 1.0000x reference)
import functools

import numpy as np
import jax
import jax.numpy as jnp
from jax import lax
from jax.experimental import pallas as pl
from jax.experimental.pallas import tpu as pltpu
from jax.experimental.pallas import tpu_sc as plsc

_NUM_CLASSES = 1000
_NROWS = 32768
_HALF = _NROWS // 2
_TEMP_LOGITS = 0.1
_NEPOCHS = 200
_TEACHER_TEMP = np.concatenate(
    (np.linspace(0.07, 0.04, 30), np.ones(_NEPOCHS - 30) * 0.04))
_RATIO = np.concatenate(
    (np.zeros(0), np.linspace(0.2, 1.0, 100), np.ones(_NEPOCHS - 0 - 100) * 1.0))
_EPOCH_FOR_RATIO = 50
_K = int(_HALF * float(_RATIO[_EPOCH_FOR_RATIO]))

_LOG2E = 1.4426950408889634

_B = 1024
_NBLK = _NROWS // _B


def _rowsum_t(m):
    ones = jnp.ones((1, _NUM_CLASSES), jnp.float32)
    return jax.lax.dot_general(
        ones, m, (((1,), (1,)), ((), ())),
        preferred_element_type=jnp.float32)


def _stage1(temp_ref, t_ref, s_ref, r_ref, d_ref, soft_ref):
    ct = jnp.float32(_LOG2E) / temp_ref[0, 0]
    cs = jnp.float32(_LOG2E / _TEMP_LOGITS)
    inv_tl = jnp.float32(1.0 / _TEMP_LOGITS)

    t = t_ref[...]
    mt1 = jnp.max(t, axis=1, keepdims=True)
    iota = jax.lax.broadcasted_iota(jnp.int32, (_B, _NUM_CLASSES), 1)
    jstar = jnp.min(jnp.where(t == mt1, iota, _NUM_CLASSES), axis=1,
                    keepdims=True)
    eqj = iota == jstar
    mt2 = jnp.max(jnp.where(eqj, -jnp.inf, t), axis=1, keepdims=True)
    e = jnp.exp2((t - mt1) * ct)

    x = s_ref[...]

    z = _rowsum_t(e)
    zs = _rowsum_t(jnp.exp2(x * cs))
    sex = _rowsum_t(e * x)
    sj = _rowsum_t(jnp.where(eqj, x, 0.0))

    eye = jnp.eye(_B, dtype=jnp.float32)
    dm = jax.lax.dot_general(
        mt2 - mt1, eye, (((0,), (0,)), ((), ())),
        preferred_element_type=jnp.float32)

    e2 = jnp.exp2(dm * ct)
    r_ref[0, :, :] = (1.0 / z) / (e2 / z + 1e-6)
    dot = sex * inv_tl / z
    lse = jnp.log(zs)
    d_ref[0, :, :] = dot - sj * inv_tl
    soft_ref[0, :, :] = lse - dot


_NTILE = 16
_PER_TILE = _HALF // _NTILE
_NVREG = _PER_TILE // 16
_UNROLL = 8


def _sc_select_body(r_hbm, d_hbm, soft_hbm, out_hbm,
                    r_v, d_v, soft_v, io_v, fio_v, iacc_v, cnt16_v,
                    acc_v, shared_v, shared_cnt):
    cid = lax.axis_index("c")
    sid = lax.axis_index("s")
    base = sid * _PER_TILE
    lane = lax.iota(jnp.int32, 16)

    pltpu.sync_copy(r_hbm.at[cid, pl.ds(base, _PER_TILE)], r_v)
    pltpu.sync_copy(d_hbm.at[cid, pl.ds(base, _PER_TILE)], d_v)
    pltpu.sync_copy(soft_hbm.at[cid, pl.ds(base, _PER_TILE)], soft_v)
    plsc.subcore_barrier()

    def count(pred):
        def blk(j, acc):
            for u in range(_UNROLL):
                off = j * (_UNROLL * 16) + u * 16
                v = r_v[pl.ds(off, 16)]
                gidx = lane + (base + off)
                acc = acc + jnp.where(pred(v, gidx), 1, 0).astype(jnp.int32)
            return acc

        return lax.fori_loop(0, _NVREG // _UNROLL, blk,
                             jnp.zeros((16,), jnp.int32))

    def global_count(slot, accvec):
        par = slot & 1
        iacc_v[pl.ds(0, 16)] = accvec
        pltpu.sync_copy(iacc_v, shared_cnt.at[par, sid])
        plsc.subcore_barrier()
        pltpu.sync_copy(shared_cnt.at[par], cnt16_v)
        tv = cnt16_v[0, pl.ds(0, 16)]
        for k in range(1, _NTILE):
            tv = tv + cnt16_v[k, pl.ds(0, 16)]
        cnt = tv[0]
        for k in range(1, 16):
            cnt = cnt + tv[k]
        return cnt

    def bs_body(it, carry):
        lo, hi = carry
        mid = lo + (hi - lo + 1) // 2
        mid_f = lax.bitcast_convert_type(mid, jnp.float32)
        total = global_count(it, count(lambda v, g: v >= mid_f))
        ge = total >= _K
        return jnp.where(ge, mid, lo), jnp.where(ge, hi, mid - 1)

    thr_i, _ = lax.fori_loop(0, 28, bs_body,
                             (jnp.int32(0x3F000000), jnp.int32(0x49F42400)))
    thr = lax.bitcast_convert_type(thr_i, jnp.float32)

    n_gt = global_count(28, count(lambda v, g: v > thr))
    need = _K - n_gt

    def tie_body(it, carry):
        lo2, hi2 = carry
        mid = (lo2 + hi2) // 2
        total = global_count(29 + it,
                             count(lambda v, g: (v == thr) & (g <= mid)))
        ok = total >= need
        return jnp.where(ok, lo2, mid + 1), jnp.where(ok, mid, hi2)

    cutoff, _ = lax.fori_loop(0, 14, tie_body,
                              (jnp.int32(0), jnp.int32(_HALF - 1)))

    def fin(j, carry):
        fa, sa = carry
        for u in range(_UNROLL):
            off = j * (_UNROLL * 16) + u * 16
            v = r_v[pl.ds(off, 16)]
            gidx = lane + (base + off)
            sel = (v > thr) | ((v == thr) & (gidx <= cutoff))
            fa = fa + jnp.where(sel, d_v[pl.ds(off, 16)], 0.0)
            sa = sa + soft_v[pl.ds(off, 16)]
        return fa, sa

    fa, sa = lax.fori_loop(0, _NVREG // _UNROLL, fin,
                           (jnp.zeros((16,), jnp.float32),
                            jnp.zeros((16,), jnp.float32)))
    fio_v[pl.ds(0, 16)] = fa + sa
    pltpu.sync_copy(fio_v, shared_v.at[sid])
    plsc.subcore_barrier()

    @pl.when(sid == 0)
    def _():
        pltpu.sync_copy(shared_v, acc_v)
        tot = acc_v[0, pl.ds(0, 16)]
        for i in range(1, _NTILE):
            tot = tot + acc_v[i, pl.ds(0, 16)]
        chunk_total = tot[0]
        for k in range(1, 16):
            chunk_total = chunk_total + tot[k]
        io_v[...] = jnp.where(lane == 0, chunk_total, 0.0)
        pltpu.sync_copy(io_v, out_hbm.at[cid])


_sc_select = functools.partial(
    pl.kernel,
    out_type=jax.ShapeDtypeStruct((2, 16), jnp.float32),
    mesh=plsc.VectorSubcoreMesh(core_axis_name="c", subcore_axis_name="s"),
    scratch_types=[
        pltpu.VMEM((_PER_TILE,), jnp.float32),
        pltpu.VMEM((_PER_TILE,), jnp.float32),
        pltpu.VMEM((_PER_TILE,), jnp.float32),
        pltpu.VMEM((16,), jnp.float32),
        pltpu.VMEM((128,), jnp.float32),
        pltpu.VMEM((128,), jnp.int32),
        pltpu.VMEM((_NTILE, 128), jnp.int32),
        pltpu.VMEM((_NTILE, 128), jnp.float32),
        pltpu.VMEM_SHARED((_NTILE, 128), jnp.float32),
        pltpu.VMEM_SHARED((2, _NTILE, 128), jnp.int32),
    ],
)(_sc_select_body)


def kernel(student_output, teacher_output, epoch):
    temp = jnp.asarray(_TEACHER_TEMP, jnp.float32)[epoch].reshape(1, 1)

    outs = pl.pallas_call(
        _stage1,
        grid=(_NBLK,),
        in_specs=[
            pl.BlockSpec((1, 1), lambda i: (0, 0), memory_space=pltpu.SMEM),
            pl.BlockSpec((_B, _NUM_CLASSES), lambda i: (i, 0)),
            pl.BlockSpec((_B, _NUM_CLASSES),
                         lambda i: ((i + _NBLK // 2) % _NBLK, 0)),
        ],
        out_specs=[pl.BlockSpec((1, 1, _B), lambda i: (i, 0, 0))] * 3,
        out_shape=[jax.ShapeDtypeStruct((_NBLK, 1, _B), jnp.float32)] * 3,
    )(temp, teacher_output, student_output)

    r2, d2, soft2 = [o.reshape(2, _HALF) for o in outs]

    totals = _sc_select(r2, d2, soft2)
    return (totals[0, 0] / _HALF + totals[1, 0] / _HALF) * 0.5
